# baseline (speedup 1.0000x reference)
import functools

import jax
import jax.numpy as jnp
from jax import lax
from jax.experimental import pallas as pl
from jax.experimental.pallas import tpu as pltpu
from jax.experimental.pallas import tpu_sc as plsc

BATCH = 16384
F = 26
D = 16
VOCAB = 100000

NC = 2
NS = 16
NW = NC * NS
TASKS = F * D
TASKS_PER_W = TASKS // NW
CB = 4096
NCH = BATCH // CB
UNROLL = 16


def _body(xT_hbm, tab_hbm, out_hbm, xv, rowv, outv0, outv1, rsem, wsem0, wsem1):
  cid = lax.axis_index("c")
  sid = lax.axis_index("s")
  wid = sid * NC + cid
  t0 = wid * TASKS_PER_W

  outvs = (outv0, outv1)
  wsems = (wsem0, wsem1)

  def row_copy(k):
    t = t0 + k
    return pltpu.make_async_copy(tab_hbm.at[t // D, t % D], rowv, rsem)

  row_copy(0).start()

  nchunks = 0
  for k in range(TASKS_PER_W):
    t = t0 + k
    f = t // D
    d = t % D

    if k == 0:
      pltpu.sync_copy(xT_hbm.at[f], xv)
    else:
      prev_f = (t - 1) // D

      @pl.when(f != prev_f)
      def _reload():
        pltpu.sync_copy(xT_hbm.at[f], xv)

    row_copy(k).wait()

    for ch in range(NCH):
      j = ch & 1
      outv = outvs[j]
      if nchunks >= 2:
        pltpu.make_async_copy(
            out_hbm.at[f, d, pl.ds(0, CB)], outv, wsems[j]).wait()

      def _gather_chunk(ch, outv):
        @plsc.parallel_loop(0, CB // 16, step=1, unroll=UNROLL)
        def _g(i):
          idx = xv[pl.ds(ch * CB + i * 16, 16)]
          outv[pl.ds(i * 16, 16)] = plsc.load_gather(rowv, [idx])

      _gather_chunk(ch, outv)
      pltpu.make_async_copy(
          outv, out_hbm.at[f, d, pl.ds(ch * CB, CB)], wsems[j]).start()
      nchunks += 1

    if k + 1 < TASKS_PER_W:
      row_copy(k + 1).start()

  tl = t0 + TASKS_PER_W - 1
  for j in range(2):
    pltpu.make_async_copy(
        out_hbm.at[tl // D, tl % D, pl.ds(0, CB)], outvs[j], wsems[j]).wait()


@jax.jit
def _gather_all(xT, tabT):
  mesh = plsc.VectorSubcoreMesh(core_axis_name="c", subcore_axis_name="s")
  kern = functools.partial(
      pl.kernel,
      mesh=mesh,
      compiler_params=pltpu.CompilerParams(
          use_tc_tiling_on_sc=True, needs_layout_passes=False),
      out_type=jax.ShapeDtypeStruct((F, D, BATCH), jnp.float32),
      scratch_types=[
          pltpu.VMEM((BATCH,), jnp.int32),
          pltpu.VMEM((VOCAB,), jnp.float32),
          pltpu.VMEM((CB,), jnp.float32),
          pltpu.VMEM((CB,), jnp.float32),
          pltpu.SemaphoreType.DMA,
          pltpu.SemaphoreType.DMA,
          pltpu.SemaphoreType.DMA,
      ],
  )(_body)
  return kern(xT, tabT)


def kernel(X_sparse, X_dense, tables):
  xT = X_sparse.T
  tabT = jnp.transpose(tables, (0, 2, 1))
  outT = _gather_all(xT, tabT)
  return jnp.transpose(outT, (2, 0, 1)), X_dense


# speedup vs baseline: 1.0239x; 1.0239x over previous
import functools

import jax
import jax.numpy as jnp
from jax import lax
from jax.experimental import pallas as pl
from jax.experimental.pallas import tpu as pltpu
from jax.experimental.pallas import tpu_sc as plsc

BATCH = 16384
F = 26
D = 16
VOCAB = 100000

NC = 2
NS = 16
NW = NC * NS
TASKS = F * D
TASKS_PER_W = TASKS // NW
CB = 4096
NCH = BATCH // CB
UNROLL = 8


def _body(xT_hbm, tab_hbm, out_hbm, xv, rowv, outv0, outv1, rsem, wsem0, wsem1):
  cid = lax.axis_index("c")
  sid = lax.axis_index("s")
  wid = sid * NC + cid
  t0 = wid * TASKS_PER_W

  outvs = (outv0, outv1)
  wsems = (wsem0, wsem1)

  def row_copy(k):
    t = t0 + k
    return pltpu.make_async_copy(tab_hbm.at[t // D, t % D], rowv, rsem)

  row_copy(0).start()

  nchunks = 0
  for k in range(TASKS_PER_W):
    t = t0 + k
    f = t // D
    d = t % D

    if k == 0:
      pltpu.sync_copy(xT_hbm.at[f], xv)
    else:
      prev_f = (t - 1) // D

      @pl.when(f != prev_f)
      def _reload():
        pltpu.sync_copy(xT_hbm.at[f], xv)

    row_copy(k).wait()

    for ch in range(NCH):
      j = ch & 1
      outv = outvs[j]
      if nchunks >= 2:
        pltpu.make_async_copy(
            out_hbm.at[f, d, pl.ds(0, CB)], outv, wsems[j]).wait()

      def _gather_chunk(ch, outv):
        @plsc.parallel_loop(0, CB // 16, step=1, unroll=UNROLL)
        def _g(i):
          idx = xv[pl.ds(ch * CB + i * 16, 16)]
          outv[pl.ds(i * 16, 16)] = plsc.load_gather(rowv, [idx])

      _gather_chunk(ch, outv)
      pltpu.make_async_copy(
          outv, out_hbm.at[f, d, pl.ds(ch * CB, CB)], wsems[j]).start()
      nchunks += 1

    if k + 1 < TASKS_PER_W:
      row_copy(k + 1).start()

  tl = t0 + TASKS_PER_W - 1
  for j in range(2):
    pltpu.make_async_copy(
        out_hbm.at[tl // D, tl % D, pl.ds(0, CB)], outvs[j], wsems[j]).wait()


@jax.jit
def _gather_all(xT, tabT):
  mesh = plsc.VectorSubcoreMesh(core_axis_name="c", subcore_axis_name="s")
  kern = functools.partial(
      pl.kernel,
      mesh=mesh,
      compiler_params=pltpu.CompilerParams(
          use_tc_tiling_on_sc=True, needs_layout_passes=False),
      out_type=jax.ShapeDtypeStruct((F, D, BATCH), jnp.float32),
      scratch_types=[
          pltpu.VMEM((BATCH,), jnp.int32),
          pltpu.VMEM((VOCAB,), jnp.float32),
          pltpu.VMEM((CB,), jnp.float32),
          pltpu.VMEM((CB,), jnp.float32),
          pltpu.SemaphoreType.DMA,
          pltpu.SemaphoreType.DMA,
          pltpu.SemaphoreType.DMA,
      ],
  )(_body)
  return kern(xT, tabT)


def kernel(X_sparse, X_dense, tables):
  xT = X_sparse.T
  tabT = jnp.transpose(tables, (0, 2, 1))
  outT = _gather_all(xT, tabT)
  return jnp.transpose(outT, (2, 0, 1)), X_dense


# speedup vs baseline: 1.0557x; 1.0311x over previous
import functools

import jax
import jax.numpy as jnp
from jax import lax
from jax.experimental import pallas as pl
from jax.experimental.pallas import tpu as pltpu
from jax.experimental.pallas import tpu_sc as plsc

BATCH = 16384
F = 26
D = 16
VOCAB = 100000

NC = 2
NS = 16
NW = NC * NS
TASKS = F * D
TASKS_PER_W = TASKS // NW
CB = 4096
NCH = BATCH // CB
UNROLL = 8


def _body(xT_hbm, tab_hbm, xd_hbm, out_hbm, od_hbm, xv, rowv, outv0, outv1,
          rsem, wsem0, wsem1):
  cid = lax.axis_index("c")
  sid = lax.axis_index("s")
  wid = sid * NC + cid
  t0 = wid * TASKS_PER_W

  outvs = (outv0, outv1)
  wsems = (wsem0, wsem1)

  def row_copy(k):
    t = t0 + k
    return pltpu.make_async_copy(tab_hbm.at[t // D, t % D], rowv, rsem)

  row_copy(0).start()

  @pl.when(wid < 13)
  def _dense():
    for ch in range(NCH):
      pltpu.sync_copy(xd_hbm.at[wid, pl.ds(ch * CB, CB)], outv0)
      pltpu.sync_copy(outv0, od_hbm.at[wid, pl.ds(ch * CB, CB)])

  nchunks = 0
  for k in range(TASKS_PER_W):
    t = t0 + k
    f = t // D
    d = t % D

    if k == 0:
      pltpu.sync_copy(xT_hbm.at[f], xv)
    else:
      prev_f = (t - 1) // D

      @pl.when(f != prev_f)
      def _reload():
        pltpu.sync_copy(xT_hbm.at[f], xv)

    row_copy(k).wait()

    for ch in range(NCH):
      j = ch & 1
      outv = outvs[j]
      if nchunks >= 2:
        pltpu.make_async_copy(
            out_hbm.at[f, d, pl.ds(0, CB)], outv, wsems[j]).wait()

      def _gather_chunk(ch, outv):
        @plsc.parallel_loop(0, CB // 16, step=1, unroll=UNROLL)
        def _g(i):
          idx = xv[pl.ds(ch * CB + i * 16, 16)]
          outv[pl.ds(i * 16, 16)] = plsc.load_gather(rowv, [idx])

      _gather_chunk(ch, outv)
      pltpu.make_async_copy(
          outv, out_hbm.at[f, d, pl.ds(ch * CB, CB)], wsems[j]).start()
      nchunks += 1

    if k + 1 < TASKS_PER_W:
      row_copy(k + 1).start()

  tl = t0 + TASKS_PER_W - 1
  for j in range(2):
    pltpu.make_async_copy(
        out_hbm.at[tl // D, tl % D, pl.ds(0, CB)], outvs[j], wsems[j]).wait()


@jax.jit
def _gather_all(xT, tabT, xdT):
  mesh = plsc.VectorSubcoreMesh(core_axis_name="c", subcore_axis_name="s")
  kern = functools.partial(
      pl.kernel,
      mesh=mesh,
      compiler_params=pltpu.CompilerParams(
          use_tc_tiling_on_sc=True, needs_layout_passes=False),
      out_type=(jax.ShapeDtypeStruct((F, D, BATCH), jnp.float32),
                jax.ShapeDtypeStruct((13, BATCH), jnp.float32)),
      scratch_types=[
          pltpu.VMEM((BATCH,), jnp.int32),
          pltpu.VMEM((VOCAB,), jnp.float32),
          pltpu.VMEM((CB,), jnp.float32),
          pltpu.VMEM((CB,), jnp.float32),
          pltpu.SemaphoreType.DMA,
          pltpu.SemaphoreType.DMA,
          pltpu.SemaphoreType.DMA,
      ],
  )(_body)
  return kern(xT, tabT, xdT)


def kernel(X_sparse, X_dense, tables):
  xT = X_sparse.T
  tabT = jnp.transpose(tables, (0, 2, 1))
  xdT = X_dense.T
  outT, odT = _gather_all(xT, tabT, xdT)
  return jnp.transpose(outT, (2, 0, 1)), odT.T


# speedup vs baseline: 1.0560x; 1.0003x over previous
import functools

import jax
import jax.numpy as jnp
from jax import lax
from jax.experimental import pallas as pl
from jax.experimental.pallas import tpu as pltpu
from jax.experimental.pallas import tpu_sc as plsc

BATCH = 16384
F = 26
D = 16
VOCAB = 100000

NC = 2
NS = 16
NW = NC * NS
TASKS = F * D
TASKS_PER_W = TASKS // NW
CB = 4096
NCH = BATCH // CB
UNROLL = 8


def _body(xT_hbm, tab_hbm, xd_hbm, out_hbm, od_hbm, xv, rowv, outv0, outv1,
          rsem, wsem0, wsem1):
  cid = lax.axis_index("c")
  sid = lax.axis_index("s")
  wid = sid * NC + cid
  t0 = wid * TASKS_PER_W

  outvs = (outv0, outv1)
  wsems = (wsem0, wsem1)

  def row_copy(k):
    t = t0 + k
    return pltpu.make_async_copy(tab_hbm.at[t // D, t % D], rowv, rsem)

  row_copy(0).start()

  @pl.when(wid < 13)
  def _dense():
    for ch in range(NCH):
      pltpu.sync_copy(xd_hbm.at[wid, pl.ds(ch * CB, CB)], outv0)
      pltpu.sync_copy(outv0, od_hbm.at[wid, pl.ds(ch * CB, CB)])

  nchunks = 0
  for k in range(TASKS_PER_W):
    t = t0 + k
    f = t // D
    d = t % D

    if k == 0:
      pltpu.sync_copy(xT_hbm.at[f], xv)
    else:
      prev_f = (t - 1) // D

      @pl.when(f != prev_f)
      def _reload():
        pltpu.sync_copy(xT_hbm.at[f], xv)

    row_copy(k).wait()

    for ch in range(NCH):
      j = ch & 1
      outv = outvs[j]
      if nchunks >= 2:
        pltpu.make_async_copy(
            out_hbm.at[f, d, pl.ds(0, CB)], outv, wsems[j]).wait()

      def _gather_chunk(ch, outv):
        @plsc.parallel_loop(0, CB // 16, step=1, unroll=UNROLL)
        def _g(i):
          idx = xv[pl.ds(ch * CB + i * 16, 16)]
          outv[pl.ds(i * 16, 16)] = plsc.load_gather(rowv, [idx])

      _gather_chunk(ch, outv)
      pltpu.make_async_copy(
          outv, out_hbm.at[f, d, pl.ds(ch * CB, CB)], wsems[j]).start()
      nchunks += 1

    if k + 1 < TASKS_PER_W:
      row_copy(k + 1).start()

  tl = t0 + TASKS_PER_W - 1
  for j in range(2):
    pltpu.make_async_copy(
        out_hbm.at[tl // D, tl % D, pl.ds(0, CB)], outvs[j], wsems[j]).wait()


@jax.jit
def _gather_all(xT, tabT, xdT):
  mesh = plsc.VectorSubcoreMesh(core_axis_name="c", subcore_axis_name="s")
  kern = functools.partial(
      pl.kernel,
      mesh=mesh,
      compiler_params=pltpu.CompilerParams(
          use_tc_tiling_on_sc=True, needs_layout_passes=False,
          disable_bounds_checks=True),
      out_type=(jax.ShapeDtypeStruct((F, D, BATCH), jnp.float32),
                jax.ShapeDtypeStruct((13, BATCH), jnp.float32)),
      scratch_types=[
          pltpu.VMEM((BATCH,), jnp.int32),
          pltpu.VMEM((VOCAB,), jnp.float32),
          pltpu.VMEM((CB,), jnp.float32),
          pltpu.VMEM((CB,), jnp.float32),
          pltpu.SemaphoreType.DMA,
          pltpu.SemaphoreType.DMA,
          pltpu.SemaphoreType.DMA,
      ],
  )(_body)
  return kern(xT, tabT, xdT)


def kernel(X_sparse, X_dense, tables):
  xT = X_sparse.T
  tabT = jnp.transpose(tables, (0, 2, 1))
  xdT = X_dense.T
  outT, odT = _gather_all(xT, tabT, xdT)
  return jnp.transpose(outT, (2, 0, 1)), odT.T
